# baseline (speedup 1.0000x reference)
import jax
import jax.numpy as jnp
from jax import lax
from jax.experimental import pallas as pl
from jax.experimental.pallas import tpu as pltpu
from jax.experimental.pallas import tpu_sc as plsc

_VOCAB = 1000000
_D = 64
_B = 4096
_N = 200

_NC = 2
_NS = 16
_NW = _NC * _NS

_BCH = 256
_NCHB = _B // _BCH
_PW = _N * _NCHB // _NW


def _body(xt_hbm, table_hbm, pos_hbm, out_hbm,
          idx0, idx1, rows0, rows1, out0, out1, pos_v, gsem, ssem):
    wid = lax.axis_index("s") * _NC + lax.axis_index("c")

    pltpu.sync_copy(pos_hbm, pos_v)
    iot = lax.iota(jnp.int32, 16)

    def coords(c):
        g = wid + _NW * c
        return g >> 4, (g & 15) * _BCH

    def stage(c, idx_v, rows_v, sbuf):
        n, b0 = coords(c)
        pltpu.sync_copy(xt_hbm.at[n].at[pl.ds(b0, _BCH)], idx_v)
        pltpu.async_copy(table_hbm.at[idx_v.at[pl.ds(0, 128)]],
                         rows_v.at[pl.ds(0, 128)], gsem.at[sbuf])
        pltpu.async_copy(table_hbm.at[idx_v.at[pl.ds(128, 128)]],
                         rows_v.at[pl.ds(128, 128)], gsem.at[sbuf])

    def wait_gather(idx_v, rows_v, sbuf):
        pltpu.make_async_copy(table_hbm.at[idx_v.at[pl.ds(0, 128)]],
                              rows_v.at[pl.ds(0, 128)], gsem.at[sbuf]).wait()
        pltpu.make_async_copy(table_hbm.at[idx_v.at[pl.ds(128, 128)]],
                              rows_v.at[pl.ds(128, 128)], gsem.at[sbuf]).wait()

    def compute(c, idx_v, rows_v, out_v, sbuf):
        wait_gather(idx_v, rows_v, sbuf)
        n, _ = coords(c)
        bp = plsc.load_gather(pos_v, [jnp.full((16,), n, jnp.int32)])

        def bl_body(bl, _):
            sl = pl.ds(bl * 16, 16)
            xcv = idx_v[sl]
            mv = jnp.where(xcv == 0, jnp.float32(0.0), jnp.float32(1.0))
            rowv = jnp.full((16,), bl * 16, jnp.int32) + iot
            for d in range(_D):
                gv = plsc.load_gather(rows_v, [rowv, jnp.full((16,), d, jnp.int32)])
                out_v[d, sl] = gv * mv + bp
            return 0

        lax.fori_loop(0, _BCH // 16, bl_body, 0)

    def scatter(c, out_v, sbuf):
        n, b0 = coords(c)
        pltpu.async_copy(out_v, out_hbm.at[n].at[:, pl.ds(b0, _BCH)], ssem.at[sbuf])

    def wait_scatter(c, out_v, sbuf):
        n, b0 = coords(c)
        pltpu.make_async_copy(out_v, out_hbm.at[n].at[:, pl.ds(b0, _BCH)],
                              ssem.at[sbuf]).wait()

    stage(0, idx0, rows0, 0)
    stage(1, idx1, rows1, 1)

    def pair_body(k, _):
        c0 = k * 2

        @pl.when(k > 0)
        def _():
            wait_scatter(c0 - 2, out0, 0)
        compute(c0, idx0, rows0, out0, 0)
        scatter(c0, out0, 0)

        @pl.when(k < _PW // 2 - 1)
        def _():
            stage(c0 + 2, idx0, rows0, 0)

        @pl.when(k > 0)
        def _():
            wait_scatter(c0 - 1, out1, 1)
        compute(c0 + 1, idx1, rows1, out1, 1)
        scatter(c0 + 1, out1, 1)

        @pl.when(k < _PW // 2 - 1)
        def _():
            stage(c0 + 3, idx1, rows1, 1)
        return 0

    lax.fori_loop(0, _PW // 2, pair_body, 0)
    wait_scatter(_PW - 2, out0, 0)
    wait_scatter(_PW - 1, out1, 1)


@jax.jit
def kernel(X, table, pos):
    xt = X.T
    table_pad = jnp.pad(table, ((0, 0), (0, _D)))
    mesh = plsc.VectorSubcoreMesh(core_axis_name="c", subcore_axis_name="s")
    out_t = pl.kernel(
        _body,
        out_type=jax.ShapeDtypeStruct((_N, _D, _B), jnp.float32),
        mesh=mesh,
        compiler_params=pltpu.CompilerParams(needs_layout_passes=False),
        scratch_types=[
            pltpu.VMEM((_BCH,), jnp.int32),
            pltpu.VMEM((_BCH,), jnp.int32),
            pltpu.VMEM((_BCH, 2 * _D), jnp.float32),
            pltpu.VMEM((_BCH, 2 * _D), jnp.float32),
            pltpu.VMEM((_D, _BCH), jnp.float32),
            pltpu.VMEM((_D, _BCH), jnp.float32),
            pltpu.VMEM((_N,), jnp.float32),
            pltpu.SemaphoreType.DMA((2,)),
            pltpu.SemaphoreType.DMA((2,)),
        ],
    )(xt, table_pad, pos[:, 0])
    return out_t.transpose(2, 0, 1)


# speedup vs baseline: 1.9504x; 1.9504x over previous
import jax
import jax.numpy as jnp
from jax import lax
from jax.experimental import pallas as pl
from jax.experimental.pallas import tpu as pltpu
from jax.experimental.pallas import tpu_sc as plsc

_VOCAB = 1000000
_D = 64
_B = 4096
_N = 200

_NC = 2
_NS = 16
_NW = _NC * _NS

_ROWS_W = _B // _NW
_NG = _N // 16
_TAIL = _NG * 16 - 8

_GATHER_DNUMS = lax.GatherDimensionNumbers(
    offset_dims=(), collapsed_slice_dims=(0,), start_index_map=(0,)
)


def _lane_broadcast(v16, j):
    idx = jnp.full((16, 1), j, jnp.int32)
    return lax.gather(
        v16, idx, _GATHER_DNUMS, (1,),
        mode=lax.GatherScatterMode.PROMISE_IN_BOUNDS,
    )


def _body(x_hbm, table_hbm, pos_hbm, out_hbm, idx_v, rows_v, out_v, pos_v,
          gsem, ssem):
    wid = lax.axis_index("s") * _NC + lax.axis_index("c")

    pltpu.sync_copy(pos_hbm, pos_v)

    def stage(c, buf):
        b = wid * _ROWS_W + c
        pltpu.sync_copy(x_hbm.at[b], idx_v.at[buf])
        pltpu.async_copy(table_hbm.at[idx_v.at[buf].at[pl.ds(0, 128)]],
                         rows_v.at[buf].at[pl.ds(0, 128)], gsem.at[buf])
        pltpu.async_copy(table_hbm.at[idx_v.at[buf].at[pl.ds(128, _N - 128)]],
                         rows_v.at[buf].at[pl.ds(128, _N - 128)], gsem.at[buf])

    def wait_gather(buf):
        pltpu.make_async_copy(table_hbm.at[idx_v.at[buf].at[pl.ds(0, 128)]],
                              rows_v.at[buf].at[pl.ds(0, 128)], gsem.at[buf]).wait()
        pltpu.make_async_copy(table_hbm.at[idx_v.at[buf].at[pl.ds(128, _N - 128)]],
                              rows_v.at[buf].at[pl.ds(128, _N - 128)],
                              gsem.at[buf]).wait()

    def compute(c, buf):
        wait_gather(buf)

        def do_rows(o, j0):
            iv16 = idx_v[buf, pl.ds(o, 16)]
            p16 = pos_v[pl.ds(o, 16)]
            m16 = jnp.where(iv16 == 0, jnp.float32(0.0), jnp.float32(1.0))
            for j in range(j0, 16):
                bm = _lane_broadcast(m16, j)
                bp = _lane_broadcast(p16, j)
                for cc in range(_D // 16):
                    v = rows_v[buf, o + j, pl.ds(cc * 16, 16)]
                    out_v[buf, o + j, pl.ds(cc * 16, 16)] = v * bm + bp

        def group_body(g, _):
            do_rows(g * 16, 0)
            return 0

        lax.fori_loop(0, _NG, group_body, 0, unroll=2)
        do_rows(_TAIL, 8)

    def scatter(c, buf):
        b = wid * _ROWS_W + c
        pltpu.async_copy(out_v.at[buf], out_hbm.at[pl.ds(b * _N, _N)], ssem.at[buf])

    def wait_scatter(c, buf):
        b = wid * _ROWS_W + c
        pltpu.make_async_copy(out_v.at[buf], out_hbm.at[pl.ds(b * _N, _N)],
                              ssem.at[buf]).wait()

    stage(0, 0)
    stage(1, 1)

    def pair_body(k, _):
        c0 = k * 2

        @pl.when(k > 0)
        def _():
            wait_scatter(c0 - 2, 0)
        compute(c0, 0)
        scatter(c0, 0)

        @pl.when(k < _ROWS_W // 2 - 1)
        def _():
            stage(c0 + 2, 0)

        @pl.when(k > 0)
        def _():
            wait_scatter(c0 - 1, 1)
        compute(c0 + 1, 1)
        scatter(c0 + 1, 1)

        @pl.when(k < _ROWS_W // 2 - 1)
        def _():
            stage(c0 + 3, 1)
        return 0

    lax.fori_loop(0, _ROWS_W // 2, pair_body, 0)
    wait_scatter(_ROWS_W - 2, 0)
    wait_scatter(_ROWS_W - 1, 1)


@jax.jit
def kernel(X, table, pos):
    table_pad = jnp.pad(table, ((0, 0), (0, _D)))
    mesh = plsc.VectorSubcoreMesh(core_axis_name="c", subcore_axis_name="s")
    out = pl.kernel(
        _body,
        out_type=jax.ShapeDtypeStruct((_B * _N, _D), jnp.float32),
        mesh=mesh,
        compiler_params=pltpu.CompilerParams(needs_layout_passes=False),
        scratch_types=[
            pltpu.VMEM((2, _N), jnp.int32),
            pltpu.VMEM((2, _N, 2 * _D), jnp.float32),
            pltpu.VMEM((2, _N, _D), jnp.float32),
            pltpu.VMEM((_N,), jnp.float32),
            pltpu.SemaphoreType.DMA((2,)),
            pltpu.SemaphoreType.DMA((2,)),
        ],
    )(X, table_pad, pos[:, 0])
    return out.reshape(_B, _N, _D)
